# Initial kernel scaffold; baseline (speedup 1.0000x reference)
import functools

import jax
import jax.numpy as jnp
from jax import lax
from jax.experimental import pallas as pl
from jax.experimental.pallas import tpu as pltpu
from jax.experimental.pallas import tpu_sc as plsc

N = 10000
E = 320000
NC = 2
NS = 16
NW = NC * NS
EPT = E // NW
CH = 125
NITER = EPT // CH
RPT = N // NS
D1 = 128
D2 = 48
BLK = 500

_mesh = plsc.VectorSubcoreMesh(
    core_axis_name="c", subcore_axis_name="s", num_cores=NC, num_subcores=NS)


@functools.partial(
    pl.kernel,
    out_type=jax.ShapeDtypeStruct((NW, N), jnp.float32),
    mesh=_mesh,
    scratch_types=[
        pltpu.VMEM((EPT,), jnp.int32),
        pltpu.VMEM((N,), jnp.float32),
    ],
)
def _deg_kernel(dst_hbm, out_hbm, idx_v, deg_v):
    c = lax.axis_index("c")
    s = lax.axis_index("s")
    wid = c * NS + s
    pltpu.sync_copy(dst_hbm.at[pl.ds(wid * EPT, EPT)], idx_v)
    zero16 = jnp.zeros((16,), jnp.float32)
    one16 = jnp.ones((16,), jnp.float32)

    def zbody(i, carry):
        deg_v[pl.ds(i * 16, 16)] = zero16
        return carry

    lax.fori_loop(0, N // 16, zbody, 0)

    def sbody(i, carry):
        idx = idx_v[pl.ds(i * 16, 16)]
        plsc.addupdate_scatter(deg_v, [idx], one16)
        return carry

    lax.fori_loop(0, EPT // 16, sbody, 0)
    pltpu.sync_copy(deg_v, out_hbm.at[wid])


def _make_agg(D):
    @functools.partial(
        pl.kernel,
        out_type=jax.ShapeDtypeStruct((NC, N, D), jnp.float32),
        mesh=_mesh,
        scratch_types=[
            pltpu.VMEM((NITER, CH), jnp.int32),
            pltpu.VMEM((NITER, CH), jnp.int32),
            pltpu.VMEM((CH, D), jnp.float32),
            pltpu.VMEM_SHARED((N, D), jnp.float32),
            pltpu.SemaphoreType.DMA,
        ],
    )
    def agg(src_hbm, dst_hbm, ht_hbm, out_hbm, isrc_v, idst_v, rows_v, acc_sh,
            sem):
        c = lax.axis_index("c")
        s = lax.axis_index("s")
        wid = c * NS + s
        pltpu.sync_copy(src_hbm.at[wid], isrc_v)
        pltpu.sync_copy(dst_hbm.at[wid], idst_v)

        zero16 = jnp.zeros((16,), jnp.float32)

        def zbody(r, carry):
            for j in range(D // 16):
                rows_v[r, pl.ds(j * 16, 16)] = zero16
            return carry

        lax.fori_loop(0, CH, zbody, 0)
        for k in range(RPT // CH):
            pltpu.sync_copy(rows_v, acc_sh.at[pl.ds(s * RPT + k * CH, CH)])
        plsc.subcore_barrier()

        def ebody(i, carry):
            pltpu.async_copy(ht_hbm.at[isrc_v.at[i]], rows_v, sem).wait()
            pltpu.sync_copy(rows_v, acc_sh.at[idst_v.at[i]], add=True)
            return carry

        lax.fori_loop(0, NITER, ebody, 0)
        plsc.subcore_barrier()
        pltpu.sync_copy(acc_sh.at[pl.ds(s * RPT, RPT)],
                        out_hbm.at[c, pl.ds(s * RPT, RPT)])

    return agg


_agg128 = _make_agg(D1)
_agg48 = _make_agg(D2)


def _stage_b_body(x_ref, w_ref, degp_ref, ht_ref, dinv_ref):
    deg = jnp.sum(degp_ref[...], axis=1, keepdims=True) + 1.0
    dinv = lax.rsqrt(deg)
    ht_ref[...] = jnp.dot(x_ref[...], w_ref[...],
                          preferred_element_type=jnp.float32) * dinv
    dinv_ref[...] = dinv


def _stage_b(x, W1, degp_t):
    return pl.pallas_call(
        _stage_b_body,
        grid=(N // BLK,),
        in_specs=[
            pl.BlockSpec((BLK, D1), lambda i: (i, 0)),
            pl.BlockSpec((D1, D1), lambda i: (0, 0)),
            pl.BlockSpec((BLK, NW), lambda i: (i, 0)),
        ],
        out_specs=[
            pl.BlockSpec((BLK, D1), lambda i: (i, 0)),
            pl.BlockSpec((BLK, 1), lambda i: (i, 0)),
        ],
        out_shape=[
            jax.ShapeDtypeStruct((N, D1), jnp.float32),
            jax.ShapeDtypeStruct((N, 1), jnp.float32),
        ],
    )(x, W1, degp_t)


def _stage_d_body(a_ref, ht_ref, dinv_ref, b_ref, w_ref, o_ref):
    acc = a_ref[0] + a_ref[1] + ht_ref[...]
    h = jnp.maximum(acc * dinv_ref[...] + b_ref[...], 0.0)
    o_ref[...] = jnp.dot(h, w_ref[...],
                         preferred_element_type=jnp.float32) * dinv_ref[...]


def _stage_d(accp, ht1, dinv, b1r, W2p):
    return pl.pallas_call(
        _stage_d_body,
        grid=(N // BLK,),
        in_specs=[
            pl.BlockSpec((NC, BLK, D1), lambda i: (0, i, 0)),
            pl.BlockSpec((BLK, D1), lambda i: (i, 0)),
            pl.BlockSpec((BLK, 1), lambda i: (i, 0)),
            pl.BlockSpec((1, D1), lambda i: (0, 0)),
            pl.BlockSpec((D1, D2), lambda i: (0, 0)),
        ],
        out_specs=pl.BlockSpec((BLK, D2), lambda i: (i, 0)),
        out_shape=jax.ShapeDtypeStruct((N, D2), jnp.float32),
    )(accp, ht1, dinv, b1r, W2p)


def _stage_f_body(a_ref, ht_ref, dinv_ref, b_ref, o_ref):
    sp = (a_ref[0] + a_ref[1] + ht_ref[...]) * dinv_ref[...] + b_ref[...]
    col = lax.broadcasted_iota(jnp.int32, (BLK, D2), 1)
    valid = col < 40
    sm = jnp.where(valid, sp, -jnp.inf)
    m = jnp.max(sm, axis=1, keepdims=True)
    e = jnp.where(valid, jnp.exp(sp - m), 0.0)
    lse = jnp.log(jnp.sum(e, axis=1, keepdims=True)) + m
    o_ref[...] = sp - lse


def _stage_f(accp2, ht2, dinv, b2r):
    return pl.pallas_call(
        _stage_f_body,
        grid=(N // BLK,),
        in_specs=[
            pl.BlockSpec((NC, BLK, D2), lambda i: (0, i, 0)),
            pl.BlockSpec((BLK, D2), lambda i: (i, 0)),
            pl.BlockSpec((BLK, 1), lambda i: (i, 0)),
            pl.BlockSpec((1, D2), lambda i: (0, 0)),
        ],
        out_specs=pl.BlockSpec((BLK, D2), lambda i: (i, 0)),
        out_shape=jax.ShapeDtypeStruct((N, D2), jnp.float32),
    )(accp2, ht2, dinv, b2r)


def kernel(x, edge_index, W1, b1, W2, b2):
    src = edge_index[0]
    dst = edge_index[1]
    src3 = src.reshape(NW, NITER, CH)
    dst3 = dst.reshape(NW, NITER, CH)

    degp = _deg_kernel(dst)
    ht1, dinv = _stage_b(x, W1, degp.T)
    accp1 = _agg128(src3, dst3, ht1)

    W2p = jnp.pad(W2, ((0, 0), (0, D2 - 40)))
    b2p = jnp.pad(b2, (0, D2 - 40)).reshape(1, D2)
    ht2 = _stage_d(accp1, ht1, dinv, b1.reshape(1, D1), W2p)
    accp2 = _agg48(src3, dst3, ht2)
    out = _stage_f(accp2, ht2, dinv, b2p)
    return out[:, :40]


# speedup vs baseline: 20.7862x; 20.7862x over previous
import functools

import jax
import jax.numpy as jnp
from jax import lax
from jax.experimental import pallas as pl
from jax.experimental.pallas import tpu as pltpu
from jax.experimental.pallas import tpu_sc as plsc

N = 10000
E = 320000
NC = 2
NS = 16
NW = NC * NS
EPT = E // NW
CH = 125
NITER = EPT // CH
EPT1 = E // NS
NITER1 = EPT1 // CH
DH = 64
ZR = 200
D1 = 128
D2 = 48
BLK = 400

_mesh = plsc.VectorSubcoreMesh(
    core_axis_name="c", subcore_axis_name="s", num_cores=NC, num_subcores=NS)
_sc_params = pltpu.CompilerParams(
    needs_layout_passes=False, use_tc_tiling_on_sc=False)


@functools.partial(
    pl.kernel,
    out_type=jax.ShapeDtypeStruct((NW, 1, N), jnp.float32),
    mesh=_mesh,
    compiler_params=_sc_params,
    scratch_types=[
        pltpu.VMEM((EPT,), jnp.int32),
        pltpu.VMEM((N,), jnp.float32),
    ],
)
def _deg_kernel(dst_hbm, out_hbm, idx_v, deg_v):
    c = lax.axis_index("c")
    s = lax.axis_index("s")
    wid = c * NS + s
    pltpu.sync_copy(dst_hbm.at[pl.ds(wid * EPT, EPT)], idx_v)
    zero16 = jnp.zeros((16,), jnp.float32)
    one16 = jnp.ones((16,), jnp.float32)

    def zbody(i, carry):
        deg_v[pl.ds(i * 16, 16)] = zero16
        return carry

    lax.fori_loop(0, N // 16, zbody, 0)

    def sbody(i, carry):
        idx = idx_v[pl.ds(i * 16, 16)]
        plsc.addupdate_scatter(deg_v, [idx], one16)
        return carry

    lax.fori_loop(0, EPT // 16, sbody, 0)
    pltpu.sync_copy(deg_v, out_hbm.at[wid, 0])


@functools.partial(
    pl.kernel,
    out_type=jax.ShapeDtypeStruct((NC, N, DH), jnp.float32),
    mesh=_mesh,
    compiler_params=_sc_params,
    scratch_types=[
        pltpu.VMEM((NITER1, CH), jnp.int32),
        pltpu.VMEM((NITER1, CH), jnp.int32),
        pltpu.VMEM((CH, DH), jnp.float32),
        pltpu.VMEM((ZR, DH), jnp.float32),
        pltpu.VMEM_SHARED((N, DH), jnp.float32),
        pltpu.SemaphoreType.DMA,
    ],
)
def _agg1(src_hbm, dst_hbm, ht_hbm, out_hbm, isrc_v, idst_v, rows_v, zbuf_v,
          acc_sh, sem):
    c = lax.axis_index("c")
    s = lax.axis_index("s")
    pltpu.sync_copy(src_hbm.at[s], isrc_v)
    pltpu.sync_copy(dst_hbm.at[s], idst_v)

    zero16 = jnp.zeros((16,), jnp.float32)

    def zbody(r, carry):
        for j in range(DH // 16):
            zbuf_v[r, pl.ds(j * 16, 16)] = zero16
        return carry

    lax.fori_loop(0, ZR, zbody, 0)

    @pl.when(s < 10)
    def _zero():
        for k in range(1000 // ZR):
            pltpu.sync_copy(zbuf_v, acc_sh.at[pl.ds(s * 1000 + k * ZR, ZR)])

    plsc.subcore_barrier()

    def ebody(i, carry):
        pltpu.async_copy(ht_hbm.at[c].at[isrc_v.at[i]], rows_v, sem).wait()
        pltpu.sync_copy(rows_v, acc_sh.at[idst_v.at[i]], add=True)
        return carry

    lax.fori_loop(0, NITER1, ebody, 0)
    plsc.subcore_barrier()

    @pl.when(s < 10)
    def _export():
        pltpu.sync_copy(acc_sh.at[pl.ds(s * 1000, 1000)],
                        out_hbm.at[c, pl.ds(s * 1000, 1000)])


def _make_agg(D):
    @functools.partial(
        pl.kernel,
        out_type=jax.ShapeDtypeStruct((NC, N, D), jnp.float32),
        mesh=_mesh,
        compiler_params=_sc_params,
        scratch_types=[
            pltpu.VMEM((NITER, CH), jnp.int32),
            pltpu.VMEM((NITER, CH), jnp.int32),
            pltpu.VMEM((CH, D), jnp.float32),
            pltpu.VMEM((ZR, D), jnp.float32),
            pltpu.VMEM_SHARED((N, D), jnp.float32),
            pltpu.SemaphoreType.DMA,
        ],
    )
    def agg(src_hbm, dst_hbm, ht_hbm, out_hbm, isrc_v, idst_v, rows_v, zbuf_v,
            acc_sh, sem):
        c = lax.axis_index("c")
        s = lax.axis_index("s")
        wid = c * NS + s
        pltpu.sync_copy(src_hbm.at[wid], isrc_v)
        pltpu.sync_copy(dst_hbm.at[wid], idst_v)

        zero16 = jnp.zeros((16,), jnp.float32)

        def zbody(r, carry):
            for j in range(D // 16):
                zbuf_v[r, pl.ds(j * 16, 16)] = zero16
            return carry

        lax.fori_loop(0, ZR, zbody, 0)

        @pl.when(s < 10)
        def _zero():
            for k in range(1000 // ZR):
                pltpu.sync_copy(zbuf_v,
                                acc_sh.at[pl.ds(s * 1000 + k * ZR, ZR)])

        plsc.subcore_barrier()

        def ebody(i, carry):
            pltpu.async_copy(ht_hbm.at[isrc_v.at[i]], rows_v, sem).wait()
            pltpu.sync_copy(rows_v, acc_sh.at[idst_v.at[i]], add=True)
            return carry

        lax.fori_loop(0, NITER, ebody, 0)
        plsc.subcore_barrier()

        @pl.when(s < 10)
        def _export():
            pltpu.sync_copy(acc_sh.at[pl.ds(s * 1000, 1000)],
                            out_hbm.at[c, pl.ds(s * 1000, 1000)])

    return agg


_agg48 = _make_agg(D2)


def _stage_b_body(x_ref, w_ref, degp_ref, ht_ref, dinv_ref):
    deg = jnp.sum(degp_ref[...], axis=1, keepdims=True) + 1.0
    dinv = lax.rsqrt(deg)
    ht = jnp.dot(x_ref[...], w_ref[...],
                 preferred_element_type=jnp.float32) * dinv
    ht_ref[0] = ht[:, :DH]
    ht_ref[1] = ht[:, DH:]
    dinv_ref[...] = dinv


def _stage_b(x, W1, degp_t):
    return pl.pallas_call(
        _stage_b_body,
        grid=(N // BLK,),
        in_specs=[
            pl.BlockSpec((BLK, D1), lambda i: (i, 0)),
            pl.BlockSpec((D1, D1), lambda i: (0, 0)),
            pl.BlockSpec((BLK, NW), lambda i: (i, 0)),
        ],
        out_specs=[
            pl.BlockSpec((NC, BLK, DH), lambda i: (0, i, 0)),
            pl.BlockSpec((BLK, 1), lambda i: (i, 0)),
        ],
        out_shape=[
            jax.ShapeDtypeStruct((NC, N, DH), jnp.float32),
            jax.ShapeDtypeStruct((N, 1), jnp.float32),
        ],
    )(x, W1, degp_t)


def _stage_d_body(a_ref, ht_ref, dinv_ref, b_ref, w_ref, o_ref):
    acc = jnp.concatenate([a_ref[0] + ht_ref[0], a_ref[1] + ht_ref[1]],
                          axis=1)
    h = jnp.maximum(acc * dinv_ref[...] + b_ref[...], 0.0)
    o_ref[...] = jnp.dot(h, w_ref[...],
                         preferred_element_type=jnp.float32) * dinv_ref[...]


def _stage_d(acc1, ht1, dinv, b1r, W2p):
    return pl.pallas_call(
        _stage_d_body,
        grid=(N // BLK,),
        in_specs=[
            pl.BlockSpec((NC, BLK, DH), lambda i: (0, i, 0)),
            pl.BlockSpec((NC, BLK, DH), lambda i: (0, i, 0)),
            pl.BlockSpec((BLK, 1), lambda i: (i, 0)),
            pl.BlockSpec((1, D1), lambda i: (0, 0)),
            pl.BlockSpec((D1, D2), lambda i: (0, 0)),
        ],
        out_specs=pl.BlockSpec((BLK, D2), lambda i: (i, 0)),
        out_shape=jax.ShapeDtypeStruct((N, D2), jnp.float32),
    )(acc1, ht1, dinv, b1r, W2p)


def _stage_f_body(a_ref, ht_ref, dinv_ref, b_ref, o_ref):
    sp = (a_ref[0] + a_ref[1] + ht_ref[...]) * dinv_ref[...] + b_ref[...]
    col = lax.broadcasted_iota(jnp.int32, (BLK, D2), 1)
    valid = col < 40
    sm = jnp.where(valid, sp, -jnp.inf)
    m = jnp.max(sm, axis=1, keepdims=True)
    e = jnp.where(valid, jnp.exp(sp - m), 0.0)
    lse = jnp.log(jnp.sum(e, axis=1, keepdims=True)) + m
    o_ref[...] = sp - lse


def _stage_f(accp2, ht2, dinv, b2r):
    return pl.pallas_call(
        _stage_f_body,
        grid=(N // BLK,),
        in_specs=[
            pl.BlockSpec((NC, BLK, D2), lambda i: (0, i, 0)),
            pl.BlockSpec((BLK, D2), lambda i: (i, 0)),
            pl.BlockSpec((BLK, 1), lambda i: (i, 0)),
            pl.BlockSpec((1, D2), lambda i: (0, 0)),
        ],
        out_specs=pl.BlockSpec((BLK, D2), lambda i: (i, 0)),
        out_shape=jax.ShapeDtypeStruct((N, D2), jnp.float32),
    )(accp2, ht2, dinv, b2r)


def kernel(x, edge_index, W1, b1, W2, b2):
    src = edge_index[0]
    dst = edge_index[1]
    src1 = src.reshape(NS, NITER1, CH)
    dst1 = dst.reshape(NS, NITER1, CH)
    src2 = src.reshape(NW, NITER, CH)
    dst2 = dst.reshape(NW, NITER, CH)

    degp = _deg_kernel(dst)
    ht1, dinv = _stage_b(x, W1, degp[:, 0, :].T)
    acc1 = _agg1(src1, dst1, ht1)

    W2p = jnp.pad(W2, ((0, 0), (0, D2 - 40)))
    b2p = jnp.pad(b2, (0, D2 - 40)).reshape(1, D2)
    ht2 = _stage_d(acc1, ht1, dinv, b1.reshape(1, D1), W2p)
    accp2 = _agg48(src2, dst2, ht2)
    out = _stage_f(accp2, ht2, dinv, b2p)
    return out[:, :40]


# speedup vs baseline: 29.7811x; 1.4327x over previous
import functools

import jax
import jax.numpy as jnp
from jax import lax
from jax.experimental import pallas as pl
from jax.experimental.pallas import tpu as pltpu
from jax.experimental.pallas import tpu_sc as plsc

N = 10000
E = 320000
NC = 2
NS = 16
NW = NC * NS
EPT = E // NW
CH = 125
NITER = EPT // CH
EPT1 = E // NS
NITER1 = EPT1 // CH
DH = 64
ZR = 200
D1 = 128
D2 = 48
BLK = 400

_mesh = plsc.VectorSubcoreMesh(
    core_axis_name="c", subcore_axis_name="s", num_cores=NC, num_subcores=NS)
_sc_params = pltpu.CompilerParams(
    needs_layout_passes=False, use_tc_tiling_on_sc=False)


def _edge_loop_db(ht_view, isrc_v, idst_v, bufs, sems, acc_sh, niter):
    pltpu.async_copy(ht_view.at[isrc_v.at[0]], bufs[0], sems[0])

    def gbody(g, carry):
        for b in range(2):
            i = g * 2 + b
            nxt = i + 1

            @pl.when(nxt < niter)
            def _start_next():
                pltpu.async_copy(ht_view.at[isrc_v.at[nxt]], bufs[1 - b],
                                 sems[1 - b])

            pltpu.make_async_copy(ht_view.at[isrc_v.at[i]], bufs[b],
                                  sems[b]).wait()
            pltpu.sync_copy(bufs[b], acc_sh.at[idst_v.at[i]], add=True)
        return carry

    lax.fori_loop(0, niter // 2, gbody, 0)


@functools.partial(
    pl.kernel,
    out_type=jax.ShapeDtypeStruct((NW, 1, N), jnp.float32),
    mesh=_mesh,
    compiler_params=_sc_params,
    scratch_types=[
        pltpu.VMEM((EPT,), jnp.int32),
        pltpu.VMEM((N,), jnp.float32),
    ],
)
def _deg_kernel(dst_hbm, out_hbm, idx_v, deg_v):
    c = lax.axis_index("c")
    s = lax.axis_index("s")
    wid = c * NS + s
    pltpu.sync_copy(dst_hbm.at[pl.ds(wid * EPT, EPT)], idx_v)
    zero16 = jnp.zeros((16,), jnp.float32)
    one16 = jnp.ones((16,), jnp.float32)

    def zbody(i, carry):
        deg_v[pl.ds(i * 16, 16)] = zero16
        return carry

    lax.fori_loop(0, N // 16, zbody, 0)

    def sbody(i, carry):
        idx = idx_v[pl.ds(i * 16, 16)]
        plsc.addupdate_scatter(deg_v, [idx], one16)
        return carry

    lax.fori_loop(0, EPT // 16, sbody, 0)
    pltpu.sync_copy(deg_v, out_hbm.at[wid, 0])


@functools.partial(
    pl.kernel,
    out_type=jax.ShapeDtypeStruct((NC, N, DH), jnp.float32),
    mesh=_mesh,
    compiler_params=_sc_params,
    scratch_types=[
        pltpu.VMEM((NITER1, CH), jnp.int32),
        pltpu.VMEM((NITER1, CH), jnp.int32),
        pltpu.VMEM((CH, DH), jnp.float32),
        pltpu.VMEM((CH, DH), jnp.float32),
        pltpu.VMEM((ZR, DH), jnp.float32),
        pltpu.VMEM_SHARED((N, DH), jnp.float32),
        pltpu.SemaphoreType.DMA,
        pltpu.SemaphoreType.DMA,
    ],
)
def _agg1(src_hbm, dst_hbm, ht_hbm, out_hbm, isrc_v, idst_v, rows0_v, rows1_v,
          zbuf_v, acc_sh, sem0, sem1):
    c = lax.axis_index("c")
    s = lax.axis_index("s")
    pltpu.sync_copy(src_hbm.at[s], isrc_v)
    pltpu.sync_copy(dst_hbm.at[s], idst_v)

    zero16 = jnp.zeros((16,), jnp.float32)

    def zbody(r, carry):
        for j in range(DH // 16):
            zbuf_v[r, pl.ds(j * 16, 16)] = zero16
        return carry

    lax.fori_loop(0, ZR, zbody, 0)

    @pl.when(s < 10)
    def _zero():
        for k in range(1000 // ZR):
            pltpu.sync_copy(zbuf_v, acc_sh.at[pl.ds(s * 1000 + k * ZR, ZR)])

    plsc.subcore_barrier()
    _edge_loop_db(ht_hbm.at[c], isrc_v, idst_v, (rows0_v, rows1_v),
                  (sem0, sem1), acc_sh, NITER1)
    plsc.subcore_barrier()

    @pl.when(s < 10)
    def _export():
        pltpu.sync_copy(acc_sh.at[pl.ds(s * 1000, 1000)],
                        out_hbm.at[c, pl.ds(s * 1000, 1000)])


def _make_agg(D):
    @functools.partial(
        pl.kernel,
        out_type=jax.ShapeDtypeStruct((NC, N, D), jnp.float32),
        mesh=_mesh,
        compiler_params=_sc_params,
        scratch_types=[
            pltpu.VMEM((NITER, CH), jnp.int32),
            pltpu.VMEM((NITER, CH), jnp.int32),
            pltpu.VMEM((CH, D), jnp.float32),
            pltpu.VMEM((CH, D), jnp.float32),
            pltpu.VMEM((ZR, D), jnp.float32),
            pltpu.VMEM_SHARED((N, D), jnp.float32),
            pltpu.SemaphoreType.DMA,
            pltpu.SemaphoreType.DMA,
        ],
    )
    def agg(src_hbm, dst_hbm, ht_hbm, out_hbm, isrc_v, idst_v, rows0_v,
            rows1_v, zbuf_v, acc_sh, sem0, sem1):
        c = lax.axis_index("c")
        s = lax.axis_index("s")
        wid = c * NS + s
        pltpu.sync_copy(src_hbm.at[wid], isrc_v)
        pltpu.sync_copy(dst_hbm.at[wid], idst_v)

        zero16 = jnp.zeros((16,), jnp.float32)

        def zbody(r, carry):
            for j in range(D // 16):
                zbuf_v[r, pl.ds(j * 16, 16)] = zero16
            return carry

        lax.fori_loop(0, ZR, zbody, 0)

        @pl.when(s < 10)
        def _zero():
            for k in range(1000 // ZR):
                pltpu.sync_copy(zbuf_v,
                                acc_sh.at[pl.ds(s * 1000 + k * ZR, ZR)])

        plsc.subcore_barrier()
        _edge_loop_db(ht_hbm, isrc_v, idst_v, (rows0_v, rows1_v),
                      (sem0, sem1), acc_sh, NITER)
        plsc.subcore_barrier()

        @pl.when(s < 10)
        def _export():
            pltpu.sync_copy(acc_sh.at[pl.ds(s * 1000, 1000)],
                            out_hbm.at[c, pl.ds(s * 1000, 1000)])

    return agg


_agg48 = _make_agg(D2)


def _stage_b_body(x_ref, w_ref, degp_ref, ht_ref, dinv_ref):
    deg = jnp.sum(degp_ref[...], axis=1, keepdims=True) + 1.0
    dinv = lax.rsqrt(deg)
    ht = jnp.dot(x_ref[...], w_ref[...],
                 preferred_element_type=jnp.float32) * dinv
    ht_ref[0] = ht[:, :DH]
    ht_ref[1] = ht[:, DH:]
    dinv_ref[...] = dinv


def _stage_b(x, W1, degp_t):
    return pl.pallas_call(
        _stage_b_body,
        grid=(N // BLK,),
        in_specs=[
            pl.BlockSpec((BLK, D1), lambda i: (i, 0)),
            pl.BlockSpec((D1, D1), lambda i: (0, 0)),
            pl.BlockSpec((BLK, NW), lambda i: (i, 0)),
        ],
        out_specs=[
            pl.BlockSpec((NC, BLK, DH), lambda i: (0, i, 0)),
            pl.BlockSpec((BLK, 1), lambda i: (i, 0)),
        ],
        out_shape=[
            jax.ShapeDtypeStruct((NC, N, DH), jnp.float32),
            jax.ShapeDtypeStruct((N, 1), jnp.float32),
        ],
    )(x, W1, degp_t)


def _stage_d_body(a_ref, ht_ref, dinv_ref, b_ref, w_ref, o_ref):
    acc = jnp.concatenate([a_ref[0] + ht_ref[0], a_ref[1] + ht_ref[1]],
                          axis=1)
    h = jnp.maximum(acc * dinv_ref[...] + b_ref[...], 0.0)
    o_ref[...] = jnp.dot(h, w_ref[...],
                         preferred_element_type=jnp.float32) * dinv_ref[...]


def _stage_d(acc1, ht1, dinv, b1r, W2p):
    return pl.pallas_call(
        _stage_d_body,
        grid=(N // BLK,),
        in_specs=[
            pl.BlockSpec((NC, BLK, DH), lambda i: (0, i, 0)),
            pl.BlockSpec((NC, BLK, DH), lambda i: (0, i, 0)),
            pl.BlockSpec((BLK, 1), lambda i: (i, 0)),
            pl.BlockSpec((1, D1), lambda i: (0, 0)),
            pl.BlockSpec((D1, D2), lambda i: (0, 0)),
        ],
        out_specs=pl.BlockSpec((BLK, D2), lambda i: (i, 0)),
        out_shape=jax.ShapeDtypeStruct((N, D2), jnp.float32),
    )(acc1, ht1, dinv, b1r, W2p)


def _stage_f_body(a_ref, ht_ref, dinv_ref, b_ref, o_ref):
    sp = (a_ref[0] + a_ref[1] + ht_ref[...]) * dinv_ref[...] + b_ref[...]
    col = lax.broadcasted_iota(jnp.int32, (BLK, D2), 1)
    valid = col < 40
    sm = jnp.where(valid, sp, -jnp.inf)
    m = jnp.max(sm, axis=1, keepdims=True)
    e = jnp.where(valid, jnp.exp(sp - m), 0.0)
    lse = jnp.log(jnp.sum(e, axis=1, keepdims=True)) + m
    o_ref[...] = sp - lse


def _stage_f(accp2, ht2, dinv, b2r):
    return pl.pallas_call(
        _stage_f_body,
        grid=(N // BLK,),
        in_specs=[
            pl.BlockSpec((NC, BLK, D2), lambda i: (0, i, 0)),
            pl.BlockSpec((BLK, D2), lambda i: (i, 0)),
            pl.BlockSpec((BLK, 1), lambda i: (i, 0)),
            pl.BlockSpec((1, D2), lambda i: (0, 0)),
        ],
        out_specs=pl.BlockSpec((BLK, D2), lambda i: (i, 0)),
        out_shape=jax.ShapeDtypeStruct((N, D2), jnp.float32),
    )(accp2, ht2, dinv, b2r)


def kernel(x, edge_index, W1, b1, W2, b2):
    src = edge_index[0]
    dst = edge_index[1]
    src1 = src.reshape(NS, NITER1, CH)
    dst1 = dst.reshape(NS, NITER1, CH)
    src2 = src.reshape(NW, NITER, CH)
    dst2 = dst.reshape(NW, NITER, CH)

    degp = _deg_kernel(dst)
    ht1, dinv = _stage_b(x, W1, degp[:, 0, :].T)
    acc1 = _agg1(src1, dst1, ht1)

    W2p = jnp.pad(W2, ((0, 0), (0, D2 - 40)))
    b2p = jnp.pad(b2, (0, D2 - 40)).reshape(1, D2)
    ht2 = _stage_d(acc1, ht1, dinv, b1.reshape(1, D1), W2p)
    accp2 = _agg48(src2, dst2, ht2)
    out = _stage_f(accp2, ht2, dinv, b2p)
    return out[:, :40]


# speedup vs baseline: 32.9942x; 1.1079x over previous
import functools

import jax
import jax.numpy as jnp
from jax import lax
from jax.experimental import pallas as pl
from jax.experimental.pallas import tpu as pltpu
from jax.experimental.pallas import tpu_sc as plsc

N = 10000
E = 320000
NC = 2
NS = 16
NW = NC * NS
EPT = E // NW
CH = 125
NITER = EPT // CH
EPT1 = E // NS
NITER1 = EPT1 // CH
DH = 64
ZR = 200
D1 = 128
D2 = 48
BLK = 2000

_mesh = plsc.VectorSubcoreMesh(
    core_axis_name="c", subcore_axis_name="s", num_cores=NC, num_subcores=NS)
_sc_params = pltpu.CompilerParams(
    needs_layout_passes=False, use_tc_tiling_on_sc=False)


def _edge_loop_db(ht_view, isrc_v, idst_v, bufs, sems, acc_sh, niter):
    pltpu.async_copy(ht_view.at[isrc_v.at[0]], bufs[0], sems[0])

    def gbody(g, carry):
        for b in range(2):
            i = g * 2 + b
            nxt = i + 1

            @pl.when(nxt < niter)
            def _start_next():
                pltpu.async_copy(ht_view.at[isrc_v.at[nxt]], bufs[1 - b],
                                 sems[1 - b])

            pltpu.make_async_copy(ht_view.at[isrc_v.at[i]], bufs[b],
                                  sems[b]).wait()
            pltpu.sync_copy(bufs[b], acc_sh.at[idst_v.at[i]], add=True)
        return carry

    lax.fori_loop(0, niter // 2, gbody, 0)


SEG = 624

@functools.partial(
    pl.kernel,
    out_type=jax.ShapeDtypeStruct((NC, N), jnp.float32),
    mesh=_mesh,
    compiler_params=_sc_params,
    scratch_types=[
        pltpu.VMEM((EPT,), jnp.int32),
        pltpu.VMEM((N,), jnp.float32),
        pltpu.VMEM((NS, SEG), jnp.float32),
        pltpu.VMEM((SEG,), jnp.float32),
        pltpu.VMEM_SHARED((NS, N), jnp.float32),
    ],
)
def _deg_kernel(edges_hbm, out_hbm, idx_v, deg_v, blk_v, red_v, hist_sh):
    c = lax.axis_index("c")
    s = lax.axis_index("s")
    wid = c * NS + s
    pltpu.sync_copy(edges_hbm.at[1, wid], idx_v)
    zero16 = jnp.zeros((16,), jnp.float32)
    one16 = jnp.ones((16,), jnp.float32)

    def zbody(i, carry):
        deg_v[pl.ds(i * 16, 16)] = zero16
        return carry

    lax.fori_loop(0, N // 16, zbody, 0)

    def sbody(i, carry):
        idx = idx_v[pl.ds(i * 16, 16)]
        plsc.addupdate_scatter(deg_v, [idx], one16)
        return carry

    lax.fori_loop(0, EPT // 16, sbody, 0)
    pltpu.sync_copy(deg_v, hist_sh.at[s])
    plsc.subcore_barrier()

    def _reduce_span(base, width):
        pltpu.sync_copy(hist_sh.at[:, pl.ds(base, width)],
                        blk_v.at[:, pl.ds(0, width)])

        def cbody(j, carry):
            off = j * 16
            v = blk_v[0, pl.ds(off, 16)]
            for r in range(1, NS):
                v = v + blk_v[r, pl.ds(off, 16)]
            red_v[pl.ds(off, 16)] = v
            return carry

        lax.fori_loop(0, width // 16, cbody, 0)
        pltpu.sync_copy(red_v.at[pl.ds(0, width)],
                        out_hbm.at[c, pl.ds(base, width)])

    _reduce_span(s * SEG, SEG)

    @pl.when(s == NS - 1)
    def _tail():
        _reduce_span(NS * SEG, N - NS * SEG)


@functools.partial(
    pl.kernel,
    out_type=jax.ShapeDtypeStruct((NC, N, DH), jnp.float32),
    mesh=_mesh,
    compiler_params=_sc_params,
    scratch_types=[
        pltpu.VMEM((NITER1, CH), jnp.int32),
        pltpu.VMEM((NITER1, CH), jnp.int32),
        pltpu.VMEM((CH, DH), jnp.float32),
        pltpu.VMEM((CH, DH), jnp.float32),
        pltpu.VMEM((ZR, DH), jnp.float32),
        pltpu.VMEM_SHARED((N, DH), jnp.float32),
        pltpu.SemaphoreType.DMA,
        pltpu.SemaphoreType.DMA,
    ],
)
def _agg1(edges_hbm, ht_hbm, out_hbm, isrc_v, idst_v, rows0_v, rows1_v,
          zbuf_v, acc_sh, sem0, sem1):
    c = lax.axis_index("c")
    s = lax.axis_index("s")
    pltpu.sync_copy(edges_hbm.at[0, s], isrc_v)
    pltpu.sync_copy(edges_hbm.at[1, s], idst_v)

    zero16 = jnp.zeros((16,), jnp.float32)

    def zbody(r, carry):
        for j in range(DH // 16):
            zbuf_v[r, pl.ds(j * 16, 16)] = zero16
        return carry

    lax.fori_loop(0, ZR, zbody, 0)

    @pl.when(s < 10)
    def _zero():
        for k in range(1000 // ZR):
            pltpu.sync_copy(zbuf_v, acc_sh.at[pl.ds(s * 1000 + k * ZR, ZR)])

    plsc.subcore_barrier()
    _edge_loop_db(ht_hbm.at[c], isrc_v, idst_v, (rows0_v, rows1_v),
                  (sem0, sem1), acc_sh, NITER1)
    plsc.subcore_barrier()

    @pl.when(s < 10)
    def _export():
        pltpu.sync_copy(acc_sh.at[pl.ds(s * 1000, 1000)],
                        out_hbm.at[c, pl.ds(s * 1000, 1000)])


def _make_agg(D):
    @functools.partial(
        pl.kernel,
        out_type=jax.ShapeDtypeStruct((NC, N, D), jnp.float32),
        mesh=_mesh,
        compiler_params=_sc_params,
        scratch_types=[
            pltpu.VMEM((NITER, CH), jnp.int32),
            pltpu.VMEM((NITER, CH), jnp.int32),
            pltpu.VMEM((CH, D), jnp.float32),
            pltpu.VMEM((CH, D), jnp.float32),
            pltpu.VMEM((ZR, D), jnp.float32),
            pltpu.VMEM_SHARED((N, D), jnp.float32),
            pltpu.SemaphoreType.DMA,
            pltpu.SemaphoreType.DMA,
        ],
    )
    def agg(edges_hbm, ht_hbm, out_hbm, isrc_v, idst_v, rows0_v,
            rows1_v, zbuf_v, acc_sh, sem0, sem1):
        c = lax.axis_index("c")
        s = lax.axis_index("s")
        wid = c * NS + s
        pltpu.sync_copy(edges_hbm.at[0, wid], isrc_v)
        pltpu.sync_copy(edges_hbm.at[1, wid], idst_v)

        zero16 = jnp.zeros((16,), jnp.float32)

        def zbody(r, carry):
            for j in range(D // 16):
                zbuf_v[r, pl.ds(j * 16, 16)] = zero16
            return carry

        lax.fori_loop(0, ZR, zbody, 0)

        @pl.when(s < 10)
        def _zero():
            for k in range(1000 // ZR):
                pltpu.sync_copy(zbuf_v,
                                acc_sh.at[pl.ds(s * 1000 + k * ZR, ZR)])

        plsc.subcore_barrier()
        _edge_loop_db(ht_hbm, isrc_v, idst_v, (rows0_v, rows1_v),
                      (sem0, sem1), acc_sh, NITER)
        plsc.subcore_barrier()

        @pl.when(s < 10)
        def _export():
            pltpu.sync_copy(acc_sh.at[pl.ds(s * 1000, 1000)],
                            out_hbm.at[c, pl.ds(s * 1000, 1000)])

    return agg


_agg48 = _make_agg(D2)


def _stage_b_body(x_ref, w_ref, degp_ref, ht_ref, dinv_ref):
    deg = jnp.sum(degp_ref[...], axis=0) + 1.0
    dinv = lax.rsqrt(deg)
    ht = jnp.dot(x_ref[...], w_ref[...],
                 preferred_element_type=jnp.float32) * dinv
    ht_ref[0] = ht[:, :DH]
    ht_ref[1] = ht[:, DH:]
    dinv_ref[...] = dinv


def _stage_b(x, W1, degp):
    return pl.pallas_call(
        _stage_b_body,
        grid=(N // BLK,),
        in_specs=[
            pl.BlockSpec((BLK, D1), lambda i: (i, 0)),
            pl.BlockSpec((D1, D1), lambda i: (0, 0)),
            pl.BlockSpec((NC, BLK, 1), lambda i: (0, i, 0)),
        ],
        out_specs=[
            pl.BlockSpec((NC, BLK, DH), lambda i: (0, i, 0)),
            pl.BlockSpec((BLK, 1), lambda i: (i, 0)),
        ],
        out_shape=[
            jax.ShapeDtypeStruct((NC, N, DH), jnp.float32),
            jax.ShapeDtypeStruct((N, 1), jnp.float32),
        ],
    )(x, W1, degp)


def _stage_d_body(a_ref, ht_ref, dinv_ref, b_ref, w_ref, o_ref):
    acc = jnp.concatenate([a_ref[0] + ht_ref[0], a_ref[1] + ht_ref[1]],
                          axis=1)
    h = jnp.maximum(acc * dinv_ref[...] + b_ref[...], 0.0)
    o_ref[...] = jnp.dot(h, w_ref[...],
                         preferred_element_type=jnp.float32) * dinv_ref[...]


def _stage_d(acc1, ht1, dinv, b1r, W2p):
    return pl.pallas_call(
        _stage_d_body,
        grid=(N // BLK,),
        in_specs=[
            pl.BlockSpec((NC, BLK, DH), lambda i: (0, i, 0)),
            pl.BlockSpec((NC, BLK, DH), lambda i: (0, i, 0)),
            pl.BlockSpec((BLK, 1), lambda i: (i, 0)),
            pl.BlockSpec((1, D1), lambda i: (0, 0)),
            pl.BlockSpec((D1, D2), lambda i: (0, 0)),
        ],
        out_specs=pl.BlockSpec((BLK, D2), lambda i: (i, 0)),
        out_shape=jax.ShapeDtypeStruct((N, D2), jnp.float32),
    )(acc1, ht1, dinv, b1r, W2p)


def _stage_f_body(a_ref, ht_ref, dinv_ref, b_ref, o_ref):
    sp = (a_ref[0] + a_ref[1] + ht_ref[...]) * dinv_ref[...] + b_ref[...]
    col = lax.broadcasted_iota(jnp.int32, (BLK, D2), 1)
    valid = col < 40
    sm = jnp.where(valid, sp, -jnp.inf)
    m = jnp.max(sm, axis=1, keepdims=True)
    e = jnp.where(valid, jnp.exp(sp - m), 0.0)
    lse = jnp.log(jnp.sum(e, axis=1, keepdims=True)) + m
    o_ref[...] = (sp - lse)[:, :40]


def _stage_f(accp2, ht2, dinv, b2r):
    return pl.pallas_call(
        _stage_f_body,
        grid=(N // BLK,),
        in_specs=[
            pl.BlockSpec((NC, BLK, D2), lambda i: (0, i, 0)),
            pl.BlockSpec((BLK, D2), lambda i: (i, 0)),
            pl.BlockSpec((BLK, 1), lambda i: (i, 0)),
            pl.BlockSpec((1, D2), lambda i: (0, 0)),
        ],
        out_specs=pl.BlockSpec((BLK, 40), lambda i: (i, 0)),
        out_shape=jax.ShapeDtypeStruct((N, 40), jnp.float32),
    )(accp2, ht2, dinv, b2r)


def kernel(x, edge_index, W1, b1, W2, b2):
    edges_d = edge_index.reshape(2, NW, EPT)
    edges_1 = edge_index.reshape(2, NS, NITER1, CH)
    edges_2 = edge_index.reshape(2, NW, NITER, CH)

    degp = _deg_kernel(edges_d)
    ht1, dinv = _stage_b(x, W1, degp.reshape(NC, N, 1))
    acc1 = _agg1(edges_1, ht1)

    W2p = jnp.pad(W2, ((0, 0), (0, D2 - 40)))
    b2p = jnp.pad(b2, (0, D2 - 40)).reshape(1, D2)
    ht2 = _stage_d(acc1, ht1, dinv, b1.reshape(1, D1), W2p)
    accp2 = _agg48(edges_2, ht2)
    return _stage_f(accp2, ht2, dinv, b2p)
